# Initial kernel scaffold; baseline (speedup 1.0000x reference)
import jax
import jax.numpy as jnp
from jax.experimental import pallas as pl


def kernel(x, edge_index, edge_attr, atom_emb, bond_emb, W1, b1, bn1_g, bn1_b, W2, b2, eps_p, bn_g, bn_b):
    raise NotImplementedError("write your pallas kernel here")


# speedup vs baseline: 1.3625x; 1.3625x over previous
import functools

import jax
import jax.numpy as jnp
from jax import lax
from jax.experimental import pallas as pl
from jax.experimental.pallas import tpu as pltpu
from jax.experimental.pallas import tpu_sc as plsc

_NC = 2
_NS = 16
_BN_EPS = 1e-5


def _cdiv(a, b):
  return -(-a // b)


def _atom_call(af, gidx, sidx, zeros, n_half, kh, d):
  zr = 8 * _cdiv(_cdiv(n_half + 1, _NS), 8)
  ah = _NS * zr
  mesh = plsc.VectorSubcoreMesh(core_axis_name="c", subcore_axis_name="s")

  @functools.partial(
      pl.kernel,
      mesh=mesh,
      out_type=jax.ShapeDtypeStruct((_NC, ah, d), jnp.float32),
      scratch_types=[
          pltpu.VMEM((kh, 128), jnp.int32),
          pltpu.VMEM((kh, 128), jnp.int32),
          pltpu.VMEM((128, d), jnp.float32),
          pltpu.VMEM_SHARED((ah, d), jnp.float32),
          pltpu.SemaphoreType.DMA,
      ],
  )
  def k(af_hbm, g_hbm, s_hbm, z_hbm, out_hbm, gi, si, buf, acc, sem):
    cid = lax.axis_index("c")
    sid = lax.axis_index("s")
    pltpu.sync_copy(z_hbm.at[pl.ds(0, zr)], acc.at[pl.ds(sid * zr, zr)])
    pltpu.sync_copy(g_hbm.at[cid, sid], gi)
    pltpu.sync_copy(s_hbm.at[cid, sid], si)
    plsc.subcore_barrier()

    @pl.loop(0, kh)
    def _(j):
      pltpu.async_copy(af_hbm.at[gi.at[j]], buf, sem).wait()
      pltpu.sync_copy(buf, acc.at[si.at[j]], add=True)

    plsc.subcore_barrier()
    pltpu.sync_copy(acc.at[pl.ds(sid * zr, zr)],
                    out_hbm.at[cid, pl.ds(sid * zr, zr)])

  return k(af, gidx, sidx, zeros)


def _edge_call(h, combo, src, dst, cidx, zeros, n, ke, d):
  zz = 8 * _cdiv(_cdiv(n + 1, _NS), 8)
  n_dum = _NS * zz
  mesh = plsc.VectorSubcoreMesh(core_axis_name="c", subcore_axis_name="s")

  @functools.partial(
      pl.kernel,
      mesh=mesh,
      out_type=jax.ShapeDtypeStruct((_NC, n_dum, d), jnp.float32),
      scratch_types=[
          pltpu.VMEM((8, 128), jnp.int32),
          pltpu.VMEM((8, 128), jnp.int32),
          pltpu.VMEM((8, 128), jnp.int32),
          pltpu.VMEM((128, d), jnp.float32),
          pltpu.VMEM((128, d), jnp.float32),
          pltpu.VMEM_SHARED((n_dum, d), jnp.float32),
          pltpu.SemaphoreType.DMA,
          pltpu.SemaphoreType.DMA,
      ],
  )
  def k(h_hbm, cmb_hbm, s_hbm, d_hbm, c_hbm, z_hbm, out_hbm,
        sv, dv, cv, hb, eb, acc, sem1, sem2):
    cid = lax.axis_index("c")
    sid = lax.axis_index("s")
    pltpu.sync_copy(z_hbm.at[pl.ds(0, zz)], acc.at[pl.ds(sid * zz, zz)])
    plsc.subcore_barrier()

    @pl.loop(0, ke // 8)
    def _(g):
      pltpu.sync_copy(s_hbm.at[cid, sid, pl.ds(g * 8, 8)], sv)
      pltpu.sync_copy(d_hbm.at[cid, sid, pl.ds(g * 8, 8)], dv)
      pltpu.sync_copy(c_hbm.at[cid, sid, pl.ds(g * 8, 8)], cv)

      @pl.loop(0, 8)
      def _(j):
        g1 = pltpu.async_copy(h_hbm.at[sv.at[j]], hb, sem1)
        g2 = pltpu.async_copy(cmb_hbm.at[cv.at[j]], eb, sem2)
        g1.wait()
        g2.wait()

        @pl.loop(0, 128)
        def _(i):
          for c in range(d // 16):
            slc = (pl.ds(i, 1), pl.ds(c * 16, 16))
            hb[slc] = jnp.maximum(hb[slc] + eb[slc], 0.0)

        pltpu.sync_copy(hb, acc.at[dv.at[j]], add=True)

    plsc.subcore_barrier()
    pltpu.sync_copy(acc.at[pl.ds(sid * zz, zz)],
                    out_hbm.at[cid, pl.ds(sid * zz, zz)])

  return k(h, combo, src, dst, cidx, zeros)


def _mlp_call(h, p, hs, w1, t1, w2, t2, relu, n, d):
  bn = 1024
  grid = (_cdiv(n, bn),)

  def body(h_ref, p_ref, hs_ref, w1_ref, t1_ref, w2_ref, t2_ref, o_ref):
    z = h_ref[...] * hs_ref[...] + p_ref[0] + p_ref[1]
    z = jnp.dot(z, w1_ref[...], preferred_element_type=jnp.float32)
    z = jnp.maximum(z + t1_ref[...], 0.0)
    z = jnp.dot(z, w2_ref[...], preferred_element_type=jnp.float32)
    z = z + t2_ref[...]
    if relu:
      z = jnp.maximum(z, 0.0)
    o_ref[...] = z

  return pl.pallas_call(
      body,
      grid=grid,
      in_specs=[
          pl.BlockSpec((bn, d), lambda i: (i, 0)),
          pl.BlockSpec((2, bn, d), lambda i: (0, i, 0)),
          pl.BlockSpec((1, d), lambda i: (0, 0)),
          pl.BlockSpec((d, d), lambda i: (0, 0)),
          pl.BlockSpec((1, d), lambda i: (0, 0)),
          pl.BlockSpec((d, d), lambda i: (0, 0)),
          pl.BlockSpec((1, d), lambda i: (0, 0)),
      ],
      out_specs=pl.BlockSpec((bn, d), lambda i: (i, 0)),
      out_shape=jax.ShapeDtypeStruct((n, d), jnp.float32),
  )(h, p, hs, w1, t1, w2, t2)


def kernel(x, edge_index, edge_attr, atom_emb, bond_emb, W1, b1, bn1_g, bn1_b,
           W2, b2, eps_p, bn_g, bn_b):
  n, nf = x.shape
  e = edge_index.shape[1]
  ma = atom_emb.shape[1]
  d = atom_emb.shape[2]
  nl = W1.shape[0]
  mb = bond_emb.shape[2]

  src = edge_index[0].astype(jnp.int32)
  dst = edge_index[1].astype(jnp.int32)
  cidx = (edge_attr[:, 0] * (mb * mb) + edge_attr[:, 1] * mb
          + edge_attr[:, 2]).astype(jnp.int32)

  gidx = ((jnp.arange(nf, dtype=jnp.int32) * ma)[None, :]
          + x.astype(jnp.int32)).reshape(-1)
  sidx = jnp.broadcast_to(jnp.arange(n, dtype=jnp.int32)[:, None],
                          (n, nf)).reshape(-1)
  n_half = n // 2
  half = n_half * nf
  kh = 8 * _cdiv(_cdiv(half, _NS * 128), 8)
  ph = kh * _NS * 128
  zrow = nf * ma

  def padh(a, fill):
    return jnp.concatenate(
        [a, jnp.full((ph - half,), fill, jnp.int32)]).reshape(_NS, kh, 128)

  gidx2 = jnp.stack([padh(gidx[:half], zrow), padh(gidx[half:], zrow)])
  sidx2 = jnp.stack([padh(sidx[:half], n_half),
                     padh(sidx[half:] - n_half, n_half)])
  af = jnp.concatenate(
      [atom_emb.reshape(nf * ma, d), jnp.zeros((1, d), jnp.float32)])

  ke = 8 * _cdiv(_cdiv(e, _NC * _NS * 128), 8)
  pe = ke * _NC * _NS * 128

  def pade(a, fill):
    return jnp.concatenate(
        [a, jnp.full((pe - e,), fill, jnp.int32)]).reshape(_NC, _NS, ke, 128)

  src2 = pade(src, 0)
  dst2 = pade(dst, n)
  cidx2 = pade(cidx, 0)

  zz = 8 * _cdiv(_cdiv(n + 1, _NS), 8)
  zeros = jnp.zeros((zz, d), jnp.float32)

  combo = (bond_emb[:, 0, :, None, None, :]
           + bond_emb[:, 1, None, :, None, :]
           + bond_emb[:, 2, None, None, :, :]).reshape(nl, mb ** 3, d)

  inv = 1.0 / jnp.sqrt(jnp.float32(1.0 + _BN_EPS))
  w1e = jnp.transpose(W1, (0, 2, 1)) * (inv * bn1_g)[:, None, :]
  t1 = (b1 * inv * bn1_g + bn1_b)[:, None, :]
  w2e = jnp.transpose(W2, (0, 2, 1)) * (inv * bn_g)[:, None, :]
  t2 = (b2 * inv * bn_g + bn_b)[:, None, :]
  hs = ((1.0 + eps_p)[:, None] * jnp.ones((1, d), jnp.float32))[:, None, :]

  pa = _atom_call(af, gidx2, sidx2, zeros, n_half, kh, d)
  h = jnp.concatenate([pa[0, :n_half], pa[1, :n_half]], axis=0)
  for l in range(nl):
    p = _edge_call(h, combo[l], src2, dst2, cidx2, zeros, n, ke, d)
    h = _mlp_call(h, p, hs[l], w1e[l], t1[l], w2e[l], t2[l],
                  relu=(l < nl - 1), n=n, d=d)
  return h


# speedup vs baseline: 3.5403x; 2.5985x over previous
import functools

import jax
import jax.numpy as jnp
from jax import lax
from jax.experimental import pallas as pl
from jax.experimental.pallas import tpu as pltpu
from jax.experimental.pallas import tpu_sc as plsc

_NC = 2
_NS = 16
_BN_EPS = 1e-5


def _cdiv(a, b):
  return -(-a // b)


def _atom_call(af, gidx, sidx, zeros, n_half, kh, d):
  zr = 8 * _cdiv(_cdiv(n_half + 1, _NS), 8)
  ah = _NS * zr
  mesh = plsc.VectorSubcoreMesh(core_axis_name="c", subcore_axis_name="s")

  @functools.partial(
      pl.kernel,
      mesh=mesh,
      out_type=jax.ShapeDtypeStruct((_NC, ah, d), jnp.float32),
      scratch_types=[
          pltpu.VMEM((kh, 128), jnp.int32),
          pltpu.VMEM((kh, 128), jnp.int32),
          pltpu.VMEM((128, d), jnp.float32),
          pltpu.VMEM((128, d), jnp.float32),
          pltpu.VMEM_SHARED((ah, d), jnp.float32),
          pltpu.SemaphoreType.DMA,
          pltpu.SemaphoreType.DMA,
      ],
  )
  def k(af_hbm, g_hbm, s_hbm, z_hbm, out_hbm, gi, si, buf0, buf1, acc,
        sem0, sem1):
    cid = lax.axis_index("c")
    sid = lax.axis_index("s")
    pltpu.sync_copy(z_hbm.at[pl.ds(0, zr)], acc.at[pl.ds(sid * zr, zr)])
    pltpu.sync_copy(g_hbm.at[cid, sid], gi)
    pltpu.sync_copy(s_hbm.at[cid, sid], si)
    plsc.subcore_barrier()

    bufs = (buf0, buf1)
    sems = (sem0, sem1)

    def gissue(j, b):
      pltpu.async_copy(af_hbm.at[gi.at[j]], bufs[b], sems[b])

    def gwait(b):
      pltpu.make_async_copy(af_hbm.at[gi.at[0]], bufs[b], sems[b]).wait()

    gissue(0, 0)

    @pl.loop(0, kh // 2)
    def _(t):
      j = 2 * t
      gwait(0)
      gissue(j + 1, 1)
      pltpu.sync_copy(buf0, acc.at[si.at[j]], add=True)
      gwait(1)

      @pl.when(t < kh // 2 - 1)
      def _():
        gissue(j + 2, 0)

      pltpu.sync_copy(buf1, acc.at[si.at[j + 1]], add=True)

    plsc.subcore_barrier()
    pltpu.sync_copy(acc.at[pl.ds(sid * zr, zr)],
                    out_hbm.at[cid, pl.ds(sid * zr, zr)])

  return k(af, gidx, sidx, zeros)


def _edge_call(h, combo, src, dst, cidx, zeros, n, kc, d):
  zz = 8 * _cdiv(_cdiv(n + 1, _NS), 8)
  n_dum = _NS * zz
  mesh = plsc.VectorSubcoreMesh(core_axis_name="c", subcore_axis_name="s")

  @functools.partial(
      pl.kernel,
      mesh=mesh,
      out_type=jax.ShapeDtypeStruct((_NC, n_dum, d), jnp.float32),
      scratch_types=[
          pltpu.VMEM((16, 64), jnp.int32),
          pltpu.VMEM((16, 64), jnp.int32),
          pltpu.VMEM((16, 64), jnp.int32),
          pltpu.VMEM((64, d), jnp.float32),
          pltpu.VMEM((64, d), jnp.float32),
          pltpu.VMEM((64, d), jnp.float32),
          pltpu.VMEM((64, d), jnp.float32),
          pltpu.VMEM_SHARED((n_dum, d), jnp.float32),
          pltpu.SemaphoreType.DMA,
          pltpu.SemaphoreType.DMA,
          pltpu.SemaphoreType.DMA,
          pltpu.SemaphoreType.DMA,
      ],
  )
  def k(h_hbm, cmb_hbm, s_hbm, d_hbm, c_hbm, z_hbm, out_hbm,
        sv, dv, cv, hb0, hb1, eb0, eb1, acc, sh0, sh1, se0, se1):
    cid = lax.axis_index("c")
    sid = lax.axis_index("s")
    pltpu.sync_copy(z_hbm.at[pl.ds(0, zz)], acc.at[pl.ds(sid * zz, zz)])
    plsc.subcore_barrier()

    hbufs, ebufs = (hb0, hb1), (eb0, eb1)
    hsems, esems = (sh0, sh1), (se0, se1)

    def gissue(j, b):
      pltpu.async_copy(h_hbm.at[sv.at[j]], hbufs[b], hsems[b])
      pltpu.async_copy(cmb_hbm.at[cv.at[j]], ebufs[b], esems[b])

    def gwait(b):
      pltpu.make_async_copy(h_hbm.at[sv.at[0]], hbufs[b], hsems[b]).wait()
      pltpu.make_async_copy(cmb_hbm.at[cv.at[0]], ebufs[b], esems[b]).wait()

    def compute(b):
      hb, eb = hbufs[b], ebufs[b]

      @pl.loop(0, 64)
      def _(i):
        for c in range(d // 16):
          slc = (pl.ds(i, 1), pl.ds(c * 16, 16))
          hb[slc] = jnp.maximum(hb[slc] + eb[slc], 0.0)

    @pl.loop(0, kc // 16)
    def _(g):
      pltpu.sync_copy(s_hbm.at[cid, sid, pl.ds(g * 16, 16)], sv)
      pltpu.sync_copy(d_hbm.at[cid, sid, pl.ds(g * 16, 16)], dv)
      pltpu.sync_copy(c_hbm.at[cid, sid, pl.ds(g * 16, 16)], cv)
      gissue(0, 0)

      @pl.loop(0, 8)
      def _(t):
        j = 2 * t
        gwait(0)
        gissue(j + 1, 1)
        compute(0)
        pltpu.sync_copy(hb0, acc.at[dv.at[j]], add=True)
        gwait(1)

        @pl.when(t < 7)
        def _():
          gissue(j + 2, 0)

        compute(1)
        pltpu.sync_copy(hb1, acc.at[dv.at[j + 1]], add=True)

    plsc.subcore_barrier()
    pltpu.sync_copy(acc.at[pl.ds(sid * zz, zz)],
                    out_hbm.at[cid, pl.ds(sid * zz, zz)])

  return k(h, combo, src, dst, cidx, zeros)


def _mlp_call(h, p, hs, w1, t1, w2, t2, relu, n, d):
  bn = 1024
  grid = (_cdiv(n, bn),)

  def body(h_ref, p_ref, hs_ref, w1_ref, t1_ref, w2_ref, t2_ref, o_ref):
    z = h_ref[...] * hs_ref[...] + p_ref[0] + p_ref[1]
    z = jnp.dot(z, w1_ref[...], preferred_element_type=jnp.float32)
    z = jnp.maximum(z + t1_ref[...], 0.0)
    z = jnp.dot(z, w2_ref[...], preferred_element_type=jnp.float32)
    z = z + t2_ref[...]
    if relu:
      z = jnp.maximum(z, 0.0)
    o_ref[...] = z

  return pl.pallas_call(
      body,
      grid=grid,
      in_specs=[
          pl.BlockSpec((bn, d), lambda i: (i, 0)),
          pl.BlockSpec((2, bn, d), lambda i: (0, i, 0)),
          pl.BlockSpec((1, d), lambda i: (0, 0)),
          pl.BlockSpec((d, d), lambda i: (0, 0)),
          pl.BlockSpec((1, d), lambda i: (0, 0)),
          pl.BlockSpec((d, d), lambda i: (0, 0)),
          pl.BlockSpec((1, d), lambda i: (0, 0)),
      ],
      out_specs=pl.BlockSpec((bn, d), lambda i: (i, 0)),
      out_shape=jax.ShapeDtypeStruct((n, d), jnp.float32),
  )(h, p, hs, w1, t1, w2, t2)


def kernel(x, edge_index, edge_attr, atom_emb, bond_emb, W1, b1, bn1_g, bn1_b,
           W2, b2, eps_p, bn_g, bn_b):
  n, nf = x.shape
  e = edge_index.shape[1]
  ma = atom_emb.shape[1]
  d = atom_emb.shape[2]
  nl = W1.shape[0]
  mb = bond_emb.shape[2]

  src = edge_index[0].astype(jnp.int32)
  dst = edge_index[1].astype(jnp.int32)
  cidx = (edge_attr[:, 0] * (mb * mb) + edge_attr[:, 1] * mb
          + edge_attr[:, 2]).astype(jnp.int32)

  gidx = ((jnp.arange(nf, dtype=jnp.int32) * ma)[None, :]
          + x.astype(jnp.int32)).reshape(-1)
  sidx = jnp.broadcast_to(jnp.arange(n, dtype=jnp.int32)[:, None],
                          (n, nf)).reshape(-1)
  n_half = n // 2
  half = n_half * nf
  kh = 8 * _cdiv(_cdiv(half, _NS * 128), 8)
  ph = kh * _NS * 128
  zrow = nf * ma

  def padh(a, fill):
    return jnp.concatenate(
        [a, jnp.full((ph - half,), fill, jnp.int32)]).reshape(_NS, kh, 128)

  gidx2 = jnp.stack([padh(gidx[:half], zrow), padh(gidx[half:], zrow)])
  sidx2 = jnp.stack([padh(sidx[:half], n_half),
                     padh(sidx[half:] - n_half, n_half)])
  af = jnp.concatenate(
      [atom_emb.reshape(nf * ma, d), jnp.zeros((1, d), jnp.float32)])

  kc = 16 * _cdiv(e, _NC * _NS * 64 * 16)
  pe = kc * _NC * _NS * 64

  def pade(a, fill):
    return jnp.concatenate(
        [a, jnp.full((pe - e,), fill, jnp.int32)]).reshape(_NC, _NS, kc, 64)

  src2 = pade(src, 0)
  dst2 = pade(dst, n)
  cidx2 = pade(cidx, 0)

  zz = 8 * _cdiv(_cdiv(n + 1, _NS), 8)
  zeros = jnp.zeros((zz, d), jnp.float32)

  combo = (bond_emb[:, 0, :, None, None, :]
           + bond_emb[:, 1, None, :, None, :]
           + bond_emb[:, 2, None, None, :, :]).reshape(nl, mb ** 3, d)

  inv = 1.0 / jnp.sqrt(jnp.float32(1.0 + _BN_EPS))
  w1e = jnp.transpose(W1, (0, 2, 1)) * (inv * bn1_g)[:, None, :]
  t1 = (b1 * inv * bn1_g + bn1_b)[:, None, :]
  w2e = jnp.transpose(W2, (0, 2, 1)) * (inv * bn_g)[:, None, :]
  t2 = (b2 * inv * bn_g + bn_b)[:, None, :]
  hs = ((1.0 + eps_p)[:, None] * jnp.ones((1, d), jnp.float32))[:, None, :]

  pa = _atom_call(af, gidx2, sidx2, zeros, n_half, kh, d)
  h = jnp.concatenate([pa[0, :n_half], pa[1, :n_half]], axis=0)
  for l in range(nl):
    p = _edge_call(h, combo[l], src2, dst2, cidx2, zeros, n, kc, d)
    h = _mlp_call(h, p, hs[l], w1e[l], t1[l], w2e[l], t2[l],
                  relu=(l < nl - 1), n=n, d=d)
  return h
